# baseline (speedup 1.0000x reference)
import functools

import jax
import jax.numpy as jnp
from jax import lax
from jax.experimental import pallas as pl
from jax.experimental.pallas import tpu as pltpu
from jax.experimental.pallas import tpu_sc as plsc

_L = 16
_NW = 32
_NP = 512
_CHUNK = (_NP * _NP) // _NW
_HNO = 8
_HH = _HNO // 2
_N1 = 501


def _prep_body(spd_ref, enc_ref, spdb_ref, tbl_ref):
    spdb_ref[...] = jnp.zeros((_NP, _NP), jnp.int32)
    spdb_ref[1 : _N1, 1 : _N1] = spd_ref[...]
    t = jnp.transpose(enc_ref[...])
    col = lax.broadcasted_iota(jnp.int32, t.shape, 1)
    t = jnp.where(col == 0, 0.0, t)
    tbl_ref[...] = jnp.pad(t, ((0, 0), (0, 64 - t.shape[1])))


def _tc_prep(spd_i, enc):
    return pl.pallas_call(
        _prep_body,
        out_shape=(
            jax.ShapeDtypeStruct((_NP, _NP), jnp.int32),
            jax.ShapeDtypeStruct((_HNO, 64), jnp.float32),
        ),
    )(spd_i, enc)


def _sc_gather_half(spd_flat, tbl_t, h0):
    mesh = plsc.VectorSubcoreMesh(core_axis_name="c", subcore_axis_name="s")

    @functools.partial(
        pl.kernel,
        mesh=mesh,
        compiler_params=pltpu.CompilerParams(needs_layout_passes=False),
        out_type=jax.ShapeDtypeStruct((_HH, _NP * _NP), jnp.float32),
        scratch_types=[
            pltpu.VMEM((_CHUNK,), jnp.int32),
            pltpu.VMEM((_HNO * 64,), jnp.float32),
            pltpu.VMEM((_HH, _CHUNK), jnp.float32),
        ],
    )
    def run(spd_hbm, tbl_hbm, out_hbm, idx_v, tbl_v, acc_v):
        wid = lax.axis_index("s") * 2 + lax.axis_index("c")
        base = wid * _CHUNK
        pltpu.sync_copy(spd_hbm.at[pl.ds(base, _CHUNK)], idx_v)
        pltpu.sync_copy(tbl_hbm, tbl_v)

        def body(k, carry):
            idx_vec = idx_v[pl.ds(k * _L, _L)]
            for h in range(_HH):
                vals = plsc.load_gather(tbl_v, [idx_vec + ((h0 + h) * 64)])
                acc_v[h, pl.ds(k * _L, _L)] = vals
            return carry

        lax.fori_loop(0, _CHUNK // _L, body, 0)
        for h in range(_HH):
            pltpu.sync_copy(acc_v.at[h], out_hbm.at[h, pl.ds(base, _CHUNK)])

    return run(spd_flat, tbl_t)


def _tc_body_a(p_ref, out_ref):
    out_ref[...] = p_ref[:, : _N1, : _N1][None]


def _tc_bcast_a(p0, B):
    return pl.pallas_call(
        _tc_body_a,
        grid=(B,),
        in_specs=[pl.BlockSpec((_HH, _NP, _NP), lambda b: (0, 0, 0))],
        out_specs=pl.BlockSpec((1, _HH, _N1, _N1), lambda b: (b, 0, 0, 0)),
        out_shape=jax.ShapeDtypeStruct((B, _HNO, _N1, _N1), jnp.float32),
    )(p0)


def _tc_body_b(prev_ref, p_ref, out_ref):
    del prev_ref
    out_ref[...] = p_ref[:, : _N1, : _N1][None]


def _tc_bcast_b(prev, p1, B):
    return pl.pallas_call(
        _tc_body_b,
        grid=(B,),
        in_specs=[
            pl.BlockSpec(memory_space=pl.ANY),
            pl.BlockSpec((_HH, _NP, _NP), lambda b: (0, 0, 0)),
        ],
        out_specs=pl.BlockSpec((1, _HH, _N1, _N1), lambda b: (b, 1, 0, 0)),
        out_shape=jax.ShapeDtypeStruct((B, _HNO, _N1, _N1), jnp.float32),
        input_output_aliases={0: 0},
    )(prev, p1)


def kernel(x, spd, sp_enc):
    B = x.shape[0]
    spd_b, tbl_t = _tc_prep(spd.astype(jnp.int32), sp_enc)
    spd_flat = spd_b.reshape(-1)
    tbl_flat = tbl_t.reshape(-1)
    p0 = _sc_gather_half(spd_flat, tbl_flat, 0)
    p1 = _sc_gather_half(spd_flat, tbl_flat, _HH)
    half = _tc_bcast_a(p0.reshape(_HH, _NP, _NP), B)
    return _tc_bcast_b(half, p1.reshape(_HH, _NP, _NP), B)


# speedup vs baseline: 1.0117x; 1.0117x over previous
import functools

import jax
import jax.numpy as jnp
from jax import lax
from jax.experimental import pallas as pl
from jax.experimental.pallas import tpu as pltpu
from jax.experimental.pallas import tpu_sc as plsc

_L = 16
_NW = 32
_NP = 512
_CHUNK = (_NP * _NP) // _NW
_HNO = 8
_N1 = 501
_TBL_PAD = 416


def _sc_gather_plane(spd_flat, tbl_flat):
    mesh = plsc.VectorSubcoreMesh(core_axis_name="c", subcore_axis_name="s")

    @functools.partial(
        pl.kernel,
        mesh=mesh,
        compiler_params=pltpu.CompilerParams(needs_layout_passes=False),
        out_type=jax.ShapeDtypeStruct((_HNO, _NP * _NP), jnp.float32),
        scratch_types=[
            pltpu.VMEM((_CHUNK,), jnp.int32),
            pltpu.VMEM((_TBL_PAD,), jnp.float32),
            pltpu.VMEM((_HNO, _CHUNK), jnp.float32),
        ],
    )
    def run(spd_hbm, tbl_hbm, out_hbm, idx_v, tbl_v, acc_v):
        wid = lax.axis_index("s") * 2 + lax.axis_index("c")
        base = wid * _CHUNK
        pltpu.sync_copy(spd_hbm.at[pl.ds(base, _CHUNK)], idx_v)
        pltpu.sync_copy(tbl_hbm, tbl_v.at[pl.ds(0, 408)])
        head = tbl_v[pl.ds(0, _L)]
        lane = lax.iota(jnp.int32, _L)
        tbl_v[pl.ds(0, _L)] = jnp.where(lane < _HNO, 0.0, head)

        def body(k, carry):
            idx8 = idx_v[pl.ds(k * _L, _L)] * _HNO
            for h in range(_HNO):
                vals = plsc.load_gather(tbl_v, [idx8 + h])
                acc_v[h, pl.ds(k * _L, _L)] = vals
            return carry

        lax.fori_loop(0, _CHUNK // _L, body, 0)
        for h in range(_HNO):
            pltpu.sync_copy(acc_v.at[h], out_hbm.at[h, pl.ds(base, _CHUNK)])

    return run(spd_flat, tbl_flat)


def _tc_body(plane_ref, out_ref):
    out_ref[...] = plane_ref[:, : _N1, : _N1][None]


def _tc_broadcast(plane, B):
    return pl.pallas_call(
        _tc_body,
        grid=(B,),
        in_specs=[pl.BlockSpec((_HNO, _NP, _NP), lambda b: (0, 0, 0))],
        out_specs=pl.BlockSpec((1, _HNO, _N1, _N1), lambda b: (b, 0, 0, 0)),
        out_shape=jax.ShapeDtypeStruct((B, _HNO, _N1, _N1), jnp.float32),
    )(plane)


def kernel(x, spd, sp_enc):
    B = x.shape[0]
    N = x.shape[2]
    spd_b = (
        jnp.zeros((_NP, _NP), jnp.int32)
        .at[1 : N + 1, 1 : N + 1]
        .set(spd.astype(jnp.int32))
    )
    plane = _sc_gather_plane(spd_b.reshape(-1), sp_enc.reshape(-1))
    return _tc_broadcast(plane.reshape(_HNO, _NP, _NP), B)


# speedup vs baseline: 1.0596x; 1.0473x over previous
import functools

import jax
import jax.numpy as jnp
from jax import lax
from jax.experimental import pallas as pl
from jax.experimental.pallas import tpu as pltpu
from jax.experimental.pallas import tpu_sc as plsc

_L = 16
_NW = 32
_NP = 512
_CHUNK = (_NP * _NP) // _NW
_HNO = 8
_N1 = 501


def _sc_gather_plane(spd_flat, tbl_t):
    mesh = plsc.VectorSubcoreMesh(core_axis_name="c", subcore_axis_name="s")

    @functools.partial(
        pl.kernel,
        mesh=mesh,
        compiler_params=pltpu.CompilerParams(needs_layout_passes=False),
        out_type=jax.ShapeDtypeStruct((_HNO, _NP * _NP), jnp.float32),
        scratch_types=[
            pltpu.VMEM((_CHUNK,), jnp.int32),
            pltpu.VMEM((_HNO * 64,), jnp.float32),
            pltpu.VMEM((_HNO, _CHUNK), jnp.float32),
        ],
    )
    def run(spd_hbm, tbl_hbm, out_hbm, idx_v, tbl_v, acc_v):
        wid = lax.axis_index("s") * 2 + lax.axis_index("c")
        base = wid * _CHUNK
        pltpu.sync_copy(spd_hbm.at[pl.ds(base, _CHUNK)], idx_v)
        pltpu.sync_copy(tbl_hbm, tbl_v)

        def body(k, carry):
            idx_vec = idx_v[pl.ds(k * _L, _L)]
            for h in range(_HNO):
                vals = plsc.load_gather(tbl_v, [idx_vec + (h * 64)])
                acc_v[h, pl.ds(k * _L, _L)] = vals
            return carry

        lax.fori_loop(0, _CHUNK // _L, body, 0)
        for h in range(_HNO):
            pltpu.sync_copy(acc_v.at[h], out_hbm.at[h, pl.ds(base, _CHUNK)])

    return run(spd_flat, tbl_t)


def _tc_body(plane_ref, out_ref):
    out_ref[...] = plane_ref[:, : _N1, : _N1][None]


def _tc_broadcast(plane, B):
    return pl.pallas_call(
        _tc_body,
        grid=(B,),
        in_specs=[pl.BlockSpec((_HNO, _NP, _NP), lambda b: (0, 0, 0))],
        out_specs=pl.BlockSpec((1, _HNO, _N1, _N1), lambda b: (b, 0, 0, 0)),
        out_shape=jax.ShapeDtypeStruct((B, _HNO, _N1, _N1), jnp.float32),
    )(plane)


def kernel(x, spd, sp_enc):
    B = x.shape[0]
    N = x.shape[2]
    table = sp_enc.at[0].set(0.0)
    tbl_t = jnp.zeros((_HNO, 64), jnp.float32).at[:, : 51].set(table.T)
    spd_b = jnp.pad(spd.astype(jnp.int32), ((1, _NP - N - 1), (1, _NP - N - 1)))
    plane = _sc_gather_plane(spd_b.reshape(-1), tbl_t.reshape(-1))
    return _tc_broadcast(plane.reshape(_HNO, _NP, _NP), B)


# speedup vs baseline: 1.1602x; 1.0950x over previous
import functools

import jax
import jax.numpy as jnp
from jax import lax
from jax.experimental import pallas as pl
from jax.experimental.pallas import tpu as pltpu
from jax.experimental.pallas import tpu_sc as plsc

_L = 16
_NW = 32
_NP = 512
_CHUNK = (_NP * _NP) // _NW
_HNO = 8
_N1 = 501


def _sc_gather_plane(spd_flat, tbl_t):
    mesh = plsc.VectorSubcoreMesh(core_axis_name="c", subcore_axis_name="s")

    @functools.partial(
        pl.kernel,
        mesh=mesh,
        compiler_params=pltpu.CompilerParams(needs_layout_passes=False),
        out_type=jax.ShapeDtypeStruct((_HNO, _NP * _NP), jnp.float32),
        scratch_types=[
            pltpu.VMEM((_CHUNK,), jnp.int32),
            pltpu.VMEM((_HNO * 64,), jnp.float32),
            pltpu.VMEM((_HNO, _CHUNK), jnp.float32),
        ],
    )
    def run(spd_hbm, tbl_hbm, out_hbm, idx_v, tbl_v, acc_v):
        wid = lax.axis_index("s") * 2 + lax.axis_index("c")
        base = wid * _CHUNK
        pltpu.sync_copy(spd_hbm.at[pl.ds(base, _CHUNK)], idx_v)
        pltpu.sync_copy(tbl_hbm, tbl_v)

        @plsc.parallel_loop(0, _CHUNK // _L, unroll=4)
        def _(k):
            idx_vec = idx_v[pl.ds(k * _L, _L)]
            for h in range(_HNO):
                vals = plsc.load_gather(tbl_v, [idx_vec + (h * 64)])
                acc_v[h, pl.ds(k * _L, _L)] = vals
        for h in range(_HNO):
            pltpu.sync_copy(acc_v.at[h], out_hbm.at[h, pl.ds(base, _CHUNK)])

    return run(spd_flat, tbl_t)


def _tc_body(plane_ref, out_ref):
    out_ref[...] = plane_ref[:, : _N1, : _N1][None]


def _tc_broadcast(plane, B):
    return pl.pallas_call(
        _tc_body,
        grid=(B,),
        in_specs=[pl.BlockSpec((_HNO, _NP, _NP), lambda b: (0, 0, 0))],
        out_specs=pl.BlockSpec((1, _HNO, _N1, _N1), lambda b: (b, 0, 0, 0)),
        out_shape=jax.ShapeDtypeStruct((B, _HNO, _N1, _N1), jnp.float32),
    )(plane)


def kernel(x, spd, sp_enc):
    B = x.shape[0]
    N = x.shape[2]
    table = sp_enc.at[0].set(0.0)
    tbl_t = jnp.zeros((_HNO, 64), jnp.float32).at[:, : 51].set(table.T)
    spd_b = jnp.pad(spd.astype(jnp.int32), ((1, _NP - N - 1), (1, _NP - N - 1)))
    plane = _sc_gather_plane(spd_b.reshape(-1), tbl_t.reshape(-1))
    return _tc_broadcast(plane.reshape(_HNO, _NP, _NP), B)
